# Initial kernel scaffold; baseline (speedup 1.0000x reference)
import jax
import jax.numpy as jnp
from jax.experimental import pallas as pl


def kernel(nodes, segment_ids, num_segments):
    raise NotImplementedError("write your pallas kernel here")


# speedup vs baseline: 3.3022x; 3.3022x over previous
import functools

import jax
import jax.numpy as jnp
from jax import lax
from jax.experimental import pallas as pl
from jax.experimental.pallas import tpu as pltpu
from jax.experimental.pallas import tpu_sc as plsc

N = 100000
D = 128
G = 64
SCALE = float(1562.5) ** (-0.5)

NC = 2
NS = 16
GROUP = 128
NFULL = N // GROUP
TAIL = N - NFULL * GROUP
DHALF = D // NC
GPW = -(-NFULL // NS)
SEGS_PER_TILE = G // NS


@functools.partial(
    pl.kernel,
    out_type=jax.ShapeDtypeStruct((G, D), jnp.float32),
    mesh=plsc.VectorSubcoreMesh(core_axis_name="c", subcore_axis_name="s"),
    compiler_params=pltpu.CompilerParams(use_tc_tiling_on_sc=False),
    scratch_types=[
        pltpu.VMEM((GROUP, DHALF), jnp.float32),
        pltpu.VMEM((GROUP,), jnp.int32),
        pltpu.VMEM((TAIL, DHALF), jnp.float32),
        pltpu.VMEM((TAIL,), jnp.int32),
        pltpu.VMEM((SEGS_PER_TILE, DHALF), jnp.float32),
        pltpu.VMEM_SHARED((G, DHALF), jnp.float32),
    ],
)
def _sc_segment_sum(nodes_ref, ids_ref, zeros_ref, out_ref,
                    buf, idxbuf, tailbuf, tailidx, outbuf, acc):
    c = lax.axis_index("c")
    s = lax.axis_index("s")
    col0 = c * DHALF

    @pl.when(s == 0)
    def _init():
        pltpu.sync_copy(zeros_ref, acc)

    plsc.subcore_barrier()

    def body(k, carry):
        g = k * NS + s

        @pl.when(g < NFULL)
        def _():
            r0 = g * GROUP
            pltpu.sync_copy(
                nodes_ref.at[pl.ds(r0, GROUP), pl.ds(col0, DHALF)], buf)
            pltpu.sync_copy(ids_ref.at[pl.ds(r0, GROUP)], idxbuf)
            pltpu.sync_copy(buf, acc.at[idxbuf], add=True)

        return carry

    lax.fori_loop(0, GPW, body, 0)

    @pl.when(s == NS - 1)
    def _tail():
        r0 = NFULL * GROUP
        pltpu.sync_copy(
            nodes_ref.at[pl.ds(r0, TAIL), pl.ds(col0, DHALF)], tailbuf)
        pltpu.sync_copy(ids_ref.at[pl.ds(r0, TAIL)], tailidx)
        pltpu.sync_copy(tailbuf, acc.at[tailidx], add=True)

    plsc.subcore_barrier()

    seg0 = s * SEGS_PER_TILE
    pltpu.sync_copy(acc.at[pl.ds(seg0, SEGS_PER_TILE)], outbuf)
    for i in range(SEGS_PER_TILE):
        for j in range(DHALF // 16):
            outbuf[i, pl.ds(j * 16, 16)] = outbuf[i, pl.ds(j * 16, 16)] * SCALE
    pltpu.sync_copy(
        outbuf, out_ref.at[pl.ds(seg0, SEGS_PER_TILE), pl.ds(col0, DHALF)])


def kernel(nodes, segment_ids, num_segments):
    ids = segment_ids.astype(jnp.int32)
    zeros = jnp.zeros((G, DHALF), jnp.float32)
    return _sc_segment_sum(nodes, ids, zeros)


# speedup vs baseline: 6.3684x; 1.9285x over previous
import functools

import jax
import jax.numpy as jnp
from jax import lax
from jax.experimental import pallas as pl
from jax.experimental.pallas import tpu as pltpu
from jax.experimental.pallas import tpu_sc as plsc

N = 100000
D = 128
G = 64
SCALE = float(1562.5) ** (-0.5)

NC = 2
NS = 16
GROUP = 128
BLOCK = 512
GPB = BLOCK // GROUP
NBLK = N // BLOCK
TAILBLK = NBLK
TAIL_ROWS = N - NBLK * BLOCK
TAIL_FULL = TAIL_ROWS // GROUP
TAIL_REM = TAIL_ROWS - TAIL_FULL * GROUP
NBLK_ALL = NBLK + 1
BPW = -(-NBLK_ALL // NS)
DHALF = D // NC
IDROWS = -(-N // GROUP) + 1
SEGS_PER_TILE = G // NS


@functools.partial(
    pl.kernel,
    out_type=jax.ShapeDtypeStruct((G, D), jnp.float32),
    mesh=plsc.VectorSubcoreMesh(core_axis_name="c", subcore_axis_name="s"),
    compiler_params=pltpu.CompilerParams(use_tc_tiling_on_sc=False),
    scratch_types=[
        pltpu.VMEM((2, BLOCK, DHALF), jnp.float32),
        pltpu.VMEM((2, GPB, GROUP), jnp.int32),
        pltpu.VMEM((TAIL_ROWS, DHALF), jnp.float32),
        pltpu.VMEM((GROUP,), jnp.int32),
        pltpu.VMEM((TAIL_REM,), jnp.int32),
        pltpu.VMEM((SEGS_PER_TILE, DHALF), jnp.float32),
        pltpu.VMEM_SHARED((G, DHALF), jnp.float32),
        pltpu.SemaphoreType.DMA,
        pltpu.SemaphoreType.DMA,
        pltpu.SemaphoreType.DMA,
        pltpu.SemaphoreType.DMA,
        pltpu.SemaphoreType.DMA,
        pltpu.SemaphoreType.DMA,
    ],
)
def _sc_segment_sum(nodes_ref, ids_ref, zeros_ref, out_ref,
                    nbuf, ibuf, tnbuf, tidx_a, tidx_b, outbuf, acc,
                    nsem0, nsem1, isem0, isem1, tnsem, tisem):
    c = lax.axis_index("c")
    s = lax.axis_index("s")
    col0 = c * DHALF
    nsems = (nsem0, nsem1)
    isems = (isem0, isem1)

    @pl.when(s == 0)
    def _init():
        pltpu.sync_copy(zeros_ref, acc)

    plsc.subcore_barrier()

    def node_copy(b, slot):
        return pltpu.make_async_copy(
            nodes_ref.at[pl.ds(b * BLOCK, BLOCK), pl.ds(col0, DHALF)],
            nbuf.at[slot], nsems[slot])

    def id_copy(b, slot):
        return pltpu.make_async_copy(
            ids_ref.at[pl.ds(b * GPB, GPB)], ibuf.at[slot], isems[slot])

    def tail_copies():
        r0 = NBLK * BLOCK
        return (
            pltpu.make_async_copy(
                nodes_ref.at[pl.ds(r0, TAIL_ROWS), pl.ds(col0, DHALF)],
                tnbuf, tnsem),
            pltpu.make_async_copy(ids_ref.at[NBLK * GPB], tidx_a, tisem),
            pltpu.make_async_copy(
                ids_ref.at[NBLK * GPB + 1, pl.ds(0, TAIL_REM)], tidx_b, tisem),
        )

    def start_load(k):
        b = k * NS + s
        slot = k % 2

        @pl.when(b < NBLK)
        def _():
            node_copy(b, slot).start()
            id_copy(b, slot).start()

        @pl.when(b == TAILBLK)
        def _():
            for cp in tail_copies():
                cp.start()

    def consume(k):
        b = k * NS + s
        slot = k % 2

        @pl.when(b < NBLK)
        def _():
            node_copy(b, slot).wait()
            id_copy(b, slot).wait()
            for j in range(GPB):
                pltpu.sync_copy(
                    nbuf.at[slot, pl.ds(j * GROUP, GROUP)],
                    acc.at[ibuf.at[slot, j]], add=True)

        @pl.when(b == TAILBLK)
        def _():
            for cp in tail_copies():
                cp.wait()
            pltpu.sync_copy(
                tnbuf.at[pl.ds(0, GROUP)], acc.at[tidx_a], add=True)
            pltpu.sync_copy(
                tnbuf.at[pl.ds(GROUP, TAIL_REM)], acc.at[tidx_b], add=True)

    start_load(0)
    for k in range(BPW):
        if k + 1 < BPW:
            start_load(k + 1)
        consume(k)

    plsc.subcore_barrier()

    seg0 = s * SEGS_PER_TILE
    pltpu.sync_copy(acc.at[pl.ds(seg0, SEGS_PER_TILE)], outbuf)
    for i in range(SEGS_PER_TILE):
        for j in range(DHALF // 16):
            outbuf[i, pl.ds(j * 16, 16)] = outbuf[i, pl.ds(j * 16, 16)] * SCALE
    pltpu.sync_copy(
        outbuf, out_ref.at[pl.ds(seg0, SEGS_PER_TILE), pl.ds(col0, DHALF)])


def kernel(nodes, segment_ids, num_segments):
    ids = segment_ids.astype(jnp.int32)
    ids = jnp.pad(ids, (0, IDROWS * GROUP - N)).reshape(IDROWS, GROUP)
    zeros = jnp.zeros((G, DHALF), jnp.float32)
    return _sc_segment_sum(nodes, ids, zeros)


# speedup vs baseline: 6.7065x; 1.0531x over previous
import functools

import jax
import jax.numpy as jnp
from jax import lax
from jax.experimental import pallas as pl
from jax.experimental.pallas import tpu as pltpu
from jax.experimental.pallas import tpu_sc as plsc

N = 100000
D = 128
G = 64
SCALE = float(1562.5) ** (-0.5)

NC = 2
NS = 16
NW = NC * NS
GROUP = 128
BLOCK = 256
GPB = BLOCK // GROUP
NBLK = N // BLOCK
TAILBLK = NBLK
TAIL_ROWS = N - NBLK * BLOCK
TAIL_REM = TAIL_ROWS - GROUP
BPW = -(-(NBLK + 1) // NW)
IDROWS = -(-N // GROUP) + 1
SEGS_PER_TILE = G // NS


@functools.partial(
    pl.kernel,
    out_type=jax.ShapeDtypeStruct((NC, G, D), jnp.float32),
    mesh=plsc.VectorSubcoreMesh(core_axis_name="c", subcore_axis_name="s"),
    compiler_params=pltpu.CompilerParams(use_tc_tiling_on_sc=False),
    scratch_types=[
        pltpu.VMEM((2, BLOCK, D), jnp.float32),
        pltpu.VMEM((2, GPB, GROUP), jnp.int32),
        pltpu.VMEM((TAIL_ROWS, D), jnp.float32),
        pltpu.VMEM((GROUP,), jnp.int32),
        pltpu.VMEM((TAIL_REM,), jnp.int32),
        pltpu.VMEM((SEGS_PER_TILE, D), jnp.float32),
        pltpu.VMEM_SHARED((G, D), jnp.float32),
        pltpu.SemaphoreType.DMA,
        pltpu.SemaphoreType.DMA,
        pltpu.SemaphoreType.DMA,
        pltpu.SemaphoreType.DMA,
        pltpu.SemaphoreType.DMA,
        pltpu.SemaphoreType.DMA,
    ],
)
def _sc_segment_sum(nodes_ref, ids_ref, zeros_ref, part_ref,
                    nbuf, ibuf, tnbuf, tidx_a, tidx_b, outbuf, acc,
                    nsem0, nsem1, isem0, isem1, tnsem, tisem):
    c = lax.axis_index("c")
    s = lax.axis_index("s")
    w = s * NC + c
    nsems = (nsem0, nsem1)
    isems = (isem0, isem1)

    @pl.when(s == 0)
    def _init():
        pltpu.sync_copy(zeros_ref, acc)

    plsc.subcore_barrier()

    def node_copy(b, slot):
        return pltpu.make_async_copy(
            nodes_ref.at[pl.ds(b * BLOCK, BLOCK)], nbuf.at[slot], nsems[slot])

    def id_copy(b, slot):
        return pltpu.make_async_copy(
            ids_ref.at[pl.ds(b * GPB, GPB)], ibuf.at[slot], isems[slot])

    def tail_copies():
        r0 = NBLK * BLOCK
        return (
            pltpu.make_async_copy(
                nodes_ref.at[pl.ds(r0, TAIL_ROWS)], tnbuf, tnsem),
            pltpu.make_async_copy(ids_ref.at[NBLK * GPB], tidx_a, tisem),
            pltpu.make_async_copy(
                ids_ref.at[NBLK * GPB + 1, pl.ds(0, TAIL_REM)], tidx_b, tisem),
        )

    def start_load(k):
        b = k * NW + w
        slot = k % 2

        @pl.when(b < NBLK)
        def _():
            node_copy(b, slot).start()
            id_copy(b, slot).start()

        @pl.when(b == TAILBLK)
        def _():
            for cp in tail_copies():
                cp.start()

    def consume(k):
        b = k * NW + w
        slot = k % 2

        @pl.when(b < NBLK)
        def _():
            node_copy(b, slot).wait()
            id_copy(b, slot).wait()
            for j in range(GPB):
                pltpu.sync_copy(
                    nbuf.at[slot, pl.ds(j * GROUP, GROUP)],
                    acc.at[ibuf.at[slot, j]], add=True)

        @pl.when(b == TAILBLK)
        def _():
            for cp in tail_copies():
                cp.wait()
            pltpu.sync_copy(
                tnbuf.at[pl.ds(0, GROUP)], acc.at[tidx_a], add=True)
            pltpu.sync_copy(
                tnbuf.at[pl.ds(GROUP, TAIL_REM)], acc.at[tidx_b], add=True)

    start_load(0)
    for k in range(BPW):
        if k + 1 < BPW:
            start_load(k + 1)
        consume(k)

    plsc.subcore_barrier()

    seg0 = s * SEGS_PER_TILE
    pltpu.sync_copy(acc.at[pl.ds(seg0, SEGS_PER_TILE)], outbuf)
    pltpu.sync_copy(outbuf, part_ref.at[c, pl.ds(seg0, SEGS_PER_TILE)])


def _combine_body(p_ref, o_ref):
    o_ref[...] = (p_ref[0] + p_ref[1]) * SCALE


def kernel(nodes, segment_ids, num_segments):
    ids = segment_ids.astype(jnp.int32)
    ids = jnp.pad(ids, (0, IDROWS * GROUP - N)).reshape(IDROWS, GROUP)
    zeros = jnp.zeros((G, D), jnp.float32)
    partials = _sc_segment_sum(nodes, ids, zeros)
    return pl.pallas_call(
        _combine_body,
        out_shape=jax.ShapeDtypeStruct((G, D), jnp.float32),
    )(partials)


# speedup vs baseline: 6.8904x; 1.0274x over previous
import functools

import jax
import jax.numpy as jnp
from jax import lax
from jax.experimental import pallas as pl
from jax.experimental.pallas import tpu as pltpu
from jax.experimental.pallas import tpu_sc as plsc

N = 100000
D = 128
G = 64
SCALE = float(1562.5) ** (-0.5)

NC = 2
NS = 16
NW = NC * NS
GROUP = 128
BLOCK = 256
GPB = BLOCK // GROUP
NBLK = N // BLOCK
TAILBLK = NBLK
TAIL_ROWS = N - NBLK * BLOCK
TAIL_REM = TAIL_ROWS - GROUP
BPW = -(-(NBLK + 1) // NW)
IDROWS = -(-N // GROUP) + 1
SEGS_PER_TILE = G // NS


@functools.partial(
    pl.kernel,
    out_type=jax.ShapeDtypeStruct((NC, G, D), jnp.float32),
    mesh=plsc.VectorSubcoreMesh(core_axis_name="c", subcore_axis_name="s"),
    compiler_params=pltpu.CompilerParams(use_tc_tiling_on_sc=False),
    scratch_types=[
        pltpu.VMEM((2, BLOCK, D), jnp.float32),
        pltpu.VMEM((2, GPB, GROUP), jnp.int32),
        pltpu.VMEM((TAIL_ROWS, D), jnp.float32),
        pltpu.VMEM((GROUP,), jnp.int32),
        pltpu.VMEM((TAIL_REM,), jnp.int32),
        pltpu.VMEM((SEGS_PER_TILE, D), jnp.float32),
        pltpu.VMEM_SHARED((G, D), jnp.float32),
        pltpu.SemaphoreType.DMA,
        pltpu.SemaphoreType.DMA,
        pltpu.SemaphoreType.DMA,
        pltpu.SemaphoreType.DMA,
        pltpu.SemaphoreType.DMA,
        pltpu.SemaphoreType.DMA,
    ],
)
def _sc_segment_sum(nodes_ref, ids_ref, zeros_ref, part_ref,
                    nbuf, ibuf, tnbuf, tidx_a, tidx_b, outbuf, acc,
                    nsem0, nsem1, isem0, isem1, tnsem, tisem):
    c = lax.axis_index("c")
    s = lax.axis_index("s")
    w = s * NC + c
    nsems = (nsem0, nsem1)
    isems = (isem0, isem1)

    @pl.when(s == 0)
    def _init():
        pltpu.sync_copy(zeros_ref, acc)

    plsc.subcore_barrier()

    def node_copy(b, slot):
        return pltpu.make_async_copy(
            nodes_ref.at[pl.ds(b * BLOCK, BLOCK)], nbuf.at[slot], nsems[slot])

    def id_copy(b, slot):
        return pltpu.make_async_copy(
            ids_ref.at[pl.ds(b * GPB, GPB)], ibuf.at[slot], isems[slot])

    def tail_copies():
        r0 = NBLK * BLOCK
        return (
            pltpu.make_async_copy(
                nodes_ref.at[pl.ds(r0, TAIL_ROWS)], tnbuf, tnsem),
            pltpu.make_async_copy(ids_ref.at[NBLK * GPB], tidx_a, tisem),
            pltpu.make_async_copy(
                ids_ref.at[NBLK * GPB + 1, pl.ds(0, TAIL_REM)], tidx_b, tisem),
        )

    def start_load(k):
        b = w * BPW + k
        slot = k % 2

        @pl.when(b < NBLK)
        def _():
            node_copy(b, slot).start()
            id_copy(b, slot).start()

        @pl.when(b == TAILBLK)
        def _():
            for cp in tail_copies():
                cp.start()

    def consume(k):
        b = w * BPW + k
        slot = k % 2

        @pl.when(b < NBLK)
        def _():
            node_copy(b, slot).wait()
            id_copy(b, slot).wait()
            for j in range(GPB):
                pltpu.sync_copy(
                    nbuf.at[slot, pl.ds(j * GROUP, GROUP)],
                    acc.at[ibuf.at[slot, j]], add=True)

        @pl.when(b == TAILBLK)
        def _():
            for cp in tail_copies():
                cp.wait()
            pltpu.sync_copy(
                tnbuf.at[pl.ds(0, GROUP)], acc.at[tidx_a], add=True)
            pltpu.sync_copy(
                tnbuf.at[pl.ds(GROUP, TAIL_REM)], acc.at[tidx_b], add=True)

    start_load(0)
    for k in range(BPW):
        if k + 1 < BPW:
            start_load(k + 1)
        consume(k)

    plsc.subcore_barrier()

    seg0 = s * SEGS_PER_TILE
    pltpu.sync_copy(acc.at[pl.ds(seg0, SEGS_PER_TILE)], outbuf)
    pltpu.sync_copy(outbuf, part_ref.at[c, pl.ds(seg0, SEGS_PER_TILE)])


def _combine_body(p_ref, o_ref):
    o_ref[...] = (p_ref[0] + p_ref[1]) * SCALE


def kernel(nodes, segment_ids, num_segments):
    ids = segment_ids.astype(jnp.int32)
    ids = jnp.pad(ids, (0, IDROWS * GROUP - N)).reshape(IDROWS, GROUP)
    zeros = jnp.zeros((G, D), jnp.float32)
    partials = _sc_segment_sum(nodes, ids, zeros)
    return pl.pallas_call(
        _combine_body,
        out_shape=jax.ShapeDtypeStruct((G, D), jnp.float32),
    )(partials)
